# Initial kernel scaffold; baseline (speedup 1.0000x reference)
from math import ceil

import jax
import jax.numpy as jnp
from jax import lax
from jax.experimental import pallas as pl
from jax.experimental.pallas import tpu as pltpu
from jax.experimental.pallas import tpu_sc as plsc

F = 128
NC = 2
NS = 16
NW = NC * NS
EPS = 1e-5
ECH = 128
NCH = 80
BR = 512
IC = 8


def _edge_agg(x_pad, src, dst, aux, n_pad):
    e2 = src.shape[0]
    epw = e2 // NW
    n_chunks = epw // ECH
    rpt = n_pad // NS

    def body(x_hbm, src_hbm, dst_hbm, aux_hbm, z_hbm, za_hbm,
             s1_out, aux_out, srcv, dstv, auxv, rowsv, s1_sh, aux_sh, sem):
        cid = lax.axis_index("c")
        sid = lax.axis_index("s")
        wid = sid * NC + cid
        row0 = sid * rpt
        pltpu.sync_copy(z_hbm.at[pl.ds(0, rpt)], s1_sh.at[pl.ds(row0, rpt)])
        pltpu.sync_copy(za_hbm.at[pl.ds(0, rpt)], aux_sh.at[pl.ds(row0, rpt)])
        plsc.subcore_barrier()
        ebase = wid * epw

        def chunk(j, carry):
            off = ebase + j * ECH
            pltpu.sync_copy(src_hbm.at[pl.ds(off, ECH)], srcv)
            pltpu.sync_copy(dst_hbm.at[pl.ds(off, ECH)], dstv)
            pltpu.sync_copy(aux_hbm.at[pl.ds(off, ECH)], auxv)
            pltpu.async_copy(x_hbm.at[srcv], rowsv, sem).wait()
            pltpu.sync_copy(rowsv, s1_sh.at[dstv], add=True)
            pltpu.sync_copy(auxv, aux_sh.at[dstv], add=True)
            return carry

        lax.fori_loop(0, n_chunks, chunk, 0)
        plsc.subcore_barrier()
        pltpu.sync_copy(s1_sh.at[pl.ds(row0, rpt)],
                        s1_out.at[cid, pl.ds(row0, rpt)])
        pltpu.sync_copy(aux_sh.at[pl.ds(row0, rpt)],
                        aux_out.at[cid, pl.ds(row0, rpt)])

    f = pl.kernel(
        body,
        out_type=(jax.ShapeDtypeStruct((NC, n_pad, F), jnp.float32),
                  jax.ShapeDtypeStruct((NC, n_pad, 16), jnp.float32)),
        mesh=plsc.VectorSubcoreMesh(core_axis_name="c", subcore_axis_name="s"),
        scratch_types=[
            pltpu.VMEM((ECH,), jnp.int32),
            pltpu.VMEM((ECH,), jnp.int32),
            pltpu.VMEM((ECH, 16), jnp.float32),
            pltpu.VMEM((ECH, F), jnp.float32),
            pltpu.VMEM_SHARED((n_pad, F), jnp.float32),
            pltpu.VMEM_SHARED((n_pad, 16), jnp.float32),
            pltpu.SemaphoreType.DMA,
        ],
    )
    zeros = jnp.zeros((rpt, F), jnp.float32)
    zerosa = jnp.zeros((rpt, 16), jnp.float32)
    return f(x_pad, src, dst, aux, zeros, zerosa)


def _conv_score(x_pad, s1p, auxp, wa, wb, we16, w1, w2, b12, wn):
    n_pad = x_pad.shape[0]

    def body(x_ref, s1_ref, aux_ref, wa_ref, wb_ref, we_ref, w1_ref, w2_ref,
             b_ref, wn_ref, cs_ref, sc_ref):
        x = x_ref[...]
        s1 = s1_ref[0] + s1_ref[1]
        aux = aux_ref[0] + aux_ref[1]
        cnt = aux[:, 4:5]
        num = (cnt * jnp.dot(x, wa_ref[...], preferred_element_type=jnp.float32)
               + jnp.dot(s1, wb_ref[...], preferred_element_type=jnp.float32)
               + jnp.dot(aux, we_ref[...], preferred_element_type=jnp.float32))
        mean = num / jnp.maximum(cnt, 1.0)
        conv = (jnp.dot(x, w1_ref[...], preferred_element_type=jnp.float32)
                + jnp.dot(mean, w2_ref[...], preferred_element_type=jnp.float32)
                + b_ref[...])
        arg = jnp.sum(conv * wn_ref[...], axis=1, keepdims=True)
        score = 1.0 / (1.0 + jnp.exp(-arg))
        cs_ref[...] = conv * score
        sc_ref[...] = score

    wspec = pl.BlockSpec((F, F), lambda i: (0, 0))
    vspec = pl.BlockSpec((1, F), lambda i: (0, 0))
    return pl.pallas_call(
        body,
        grid=(n_pad // BR,),
        in_specs=[
            pl.BlockSpec((BR, F), lambda i: (i, 0)),
            pl.BlockSpec((NC, BR, F), lambda i: (0, i, 0)),
            pl.BlockSpec((NC, BR, 16), lambda i: (0, i, 0)),
            wspec, wspec,
            pl.BlockSpec((16, F), lambda i: (0, 0)),
            wspec, wspec, vspec, vspec,
        ],
        out_specs=[pl.BlockSpec((BR, F), lambda i: (i, 0)),
                   pl.BlockSpec((BR, 1), lambda i: (i, 0))],
        out_shape=[jax.ShapeDtypeStruct((n_pad, F), jnp.float32),
                   jax.ShapeDtypeStruct((n_pad, 1), jnp.float32)],
    )(x_pad, s1p, auxp, wa, wb, we16, w1, w2, b12, wn)


def _rank(score2d, n_valid, k):
    g = score2d.shape[0]

    def body(s_ref, rank_ref, nmap_ref):
        i0 = pl.program_id(0) * IC
        s = s_ref[...]
        gidx = (lax.broadcasted_iota(jnp.int32, (g, 128), 0) * 128
                + lax.broadcasted_iota(jnp.int32, (g, 128), 1))
        sv = jnp.where(gidx < n_valid, s, -jnp.inf)
        si = lax.dynamic_slice(sv, (i0, 0), (IC, 128))[:, :, None]
        ii = lax.dynamic_slice(gidx, (i0, 0), (IC, 128))[:, :, None]

        def jstep(jc, acc):
            sj = lax.dynamic_slice(sv, (jc, 0), (1, 128))[None]
            jj = (jc * 128
                  + lax.broadcasted_iota(jnp.int32, (1, 1, 128), 2))
            cmp = (sj > si) | ((sj == si) & (jj < ii))
            return acc + jnp.sum(cmp.astype(jnp.int32), axis=2)

        acc = lax.fori_loop(0, g, jstep, jnp.zeros((IC, 128), jnp.int32))
        iival = ii[:, :, 0]
        valid_i = iival < n_valid
        rank = jnp.where(valid_i, acc, iival)
        rank_ref[...] = rank
        nmap_ref[...] = jnp.where(valid_i & (rank < k), rank, -1)

    return pl.pallas_call(
        body,
        grid=(g // IC,),
        in_specs=[pl.BlockSpec((g, 128), lambda i: (0, 0))],
        out_specs=[pl.BlockSpec((IC, 128), lambda i: (i, 0)),
                   pl.BlockSpec((IC, 128), lambda i: (i, 0))],
        out_shape=[jax.ShapeDtypeStruct((g, 128), jnp.int32),
                   jax.ShapeDtypeStruct((g, 128), jnp.int32)],
    )(score2d)


def _permute(cs, rank, n_pad, remap=None):
    npt = n_pad // NW
    n_iters = npt // NCH
    do_remap = remap is not None
    if do_remap:
        nmap, src, dst, n_new = remap
        e2 = src.shape[0]
        epw = e2 // NW
        e_chunks = epw // ECH

    def body(*refs):
        if do_remap:
            (cs_hbm, rank_hbm, nmap_hbm, src_hbm, dst_hbm, xout, src2, dst2,
             rowsv, rkv, nmapv, srcv, dstv, osrcv, odstv, sem) = refs
        else:
            cs_hbm, rank_hbm, xout, rowsv, rkv, sem = refs
        cid = lax.axis_index("c")
        sid = lax.axis_index("s")
        wid = sid * NC + cid
        nb0 = wid * npt

        def pchunk(j, carry):
            nb = nb0 + j * NCH
            pltpu.sync_copy(cs_hbm.at[pl.ds(nb, NCH)], rowsv)
            pltpu.sync_copy(rank_hbm.at[pl.ds(nb, NCH)], rkv)
            pltpu.async_copy(rowsv, xout.at[rkv], sem).wait()
            return carry

        lax.fori_loop(0, n_iters, pchunk, 0)

        if do_remap:
            pltpu.sync_copy(nmap_hbm, nmapv)
            ebase = wid * epw

            def echunk(j, carry):
                off = ebase + j * ECH
                pltpu.sync_copy(src_hbm.at[pl.ds(off, ECH)], srcv)
                pltpu.sync_copy(dst_hbm.at[pl.ds(off, ECH)], dstv)
                for t in range(ECH // 16):
                    sl = pl.ds(t * 16, 16)
                    a = plsc.load_gather(nmapv, [srcv[sl]])
                    b = plsc.load_gather(nmapv, [dstv[sl]])
                    ok = (a >= 0) & (b >= 0)
                    gid = off + t * 16 + lax.iota(jnp.int32, 16)
                    osrcv[sl] = jnp.where(ok, a, gid & 4095)
                    odstv[sl] = jnp.where(ok, b, n_new + (gid & 63))
                pltpu.sync_copy(osrcv, src2.at[pl.ds(off, ECH)])
                pltpu.sync_copy(odstv, dst2.at[pl.ds(off, ECH)])
                return carry

            lax.fori_loop(0, e_chunks, echunk, 0)

    out_type = [jax.ShapeDtypeStruct((n_pad, F), jnp.float32)]
    scratch = [
        pltpu.VMEM((NCH, F), jnp.float32),
        pltpu.VMEM((NCH,), jnp.int32),
    ]
    args = [cs, rank]
    if do_remap:
        out_type += [jax.ShapeDtypeStruct((e2,), jnp.int32),
                     jax.ShapeDtypeStruct((e2,), jnp.int32)]
        scratch += [
            pltpu.VMEM((n_pad,), jnp.int32),
            pltpu.VMEM((ECH,), jnp.int32),
            pltpu.VMEM((ECH,), jnp.int32),
            pltpu.VMEM((ECH,), jnp.int32),
            pltpu.VMEM((ECH,), jnp.int32),
        ]
        args += [nmap, src, dst]
    scratch += [pltpu.SemaphoreType.DMA]

    f = pl.kernel(
        body,
        out_type=tuple(out_type),
        mesh=plsc.VectorSubcoreMesh(core_axis_name="c", subcore_axis_name="s"),
        scratch_types=scratch,
    )
    out = f(*args)
    return out if do_remap else (out,)


def _bn_pool(x_sel, kk, npk, gamma, beta):
    def body(x_ref, g_ref, b_ref, xbn_ref, pool_ref):
        xp = x_ref[...]
        m = lax.broadcasted_iota(jnp.int32, (npk, 1), 0) < kk
        xm = jnp.where(m, xp, 0.0)
        mu = jnp.sum(xm, axis=0, keepdims=True) / kk
        var = jnp.sum(xm * xm, axis=0, keepdims=True) / kk - mu * mu
        inv = lax.rsqrt(var + EPS)
        xn = jnp.maximum((xp - mu) * inv * g_ref[...] + b_ref[...], 0.0)
        xn = jnp.where(m, xn, 0.0)
        xbn_ref[...] = xn
        mx = jnp.max(jnp.where(m, xn, -jnp.inf), axis=0, keepdims=True)
        mn = jnp.sum(xn, axis=0, keepdims=True) / kk
        pool_ref[...] = jnp.concatenate([mx, mn], axis=1)

    return pl.pallas_call(
        body,
        in_specs=[pl.BlockSpec((npk, F), lambda: (0, 0)),
                  pl.BlockSpec((1, F), lambda: (0, 0)),
                  pl.BlockSpec((1, F), lambda: (0, 0))],
        out_specs=[pl.BlockSpec((npk, F), lambda: (0, 0)),
                   pl.BlockSpec((1, 2 * F), lambda: (0, 0))],
        out_shape=[jax.ShapeDtypeStruct((npk, F), jnp.float32),
                   jax.ShapeDtypeStruct((1, 2 * F), jnp.float32)],
    )(x_sel, gamma, beta)


def _fold_weights(p):
    wa = p['Wpre'][0:F]
    wb = p['Wpre'][F:2 * F]
    c = p['Wpre'][2 * F:3 * F]
    we16 = (jnp.zeros((16, F), jnp.float32)
            .at[0:4].set(p['We'] @ c)
            .at[4].set(p['be'] @ c + p['bpre']))
    w1 = p['Wpost'][:F] @ p['Wlin']
    w2 = p['Wpost'][F:] @ p['Wlin']
    b12 = (p['bpost'] @ p['Wlin'] + p['blin'])[None]
    wn = (p['wpool'] / jnp.linalg.norm(p['wpool']))[None]
    return wa, wb, we16, w1, w2, b12, wn


def _round_up(v, m):
    return ((v + m - 1) // m) * m


def kernel(x, edge_index, edge_attr, batch, params):
    n1 = x.shape[0]
    e = edge_index.shape[1]
    k1 = int(ceil(0.5 * n1))
    n2 = k1
    k2 = int(ceil(0.5 * n2))
    npad1 = _round_up(n1 + 64, NW * NCH)
    npad2 = _round_up(n2 + 64, NW * NCH)
    npk1 = _round_up(k1, NW * NCH)
    npk2 = _round_up(k2, NW * NCH)
    e2 = _round_up(e, NW * ECH)

    src = edge_index[0]
    dst = edge_index[1]
    pad_e = e2 - e
    eid = jnp.arange(pad_e, dtype=jnp.int32)
    src_p = jnp.concatenate([src, eid & 4095])
    dst_p = jnp.concatenate([dst, n1 + (eid & 63)])
    aux = (jnp.zeros((e2, 16), jnp.float32)
           .at[:e, :4].set(edge_attr)
           .at[:e, 4].set(1.0))
    x_pad = jnp.zeros((npad1, F), jnp.float32).at[:n1].set(x)

    wl1 = _fold_weights(params[0])
    wl2 = _fold_weights(params[1])

    s1p, auxp = _edge_agg(x_pad, src_p, dst_p, aux, npad1)
    cs, score = _conv_score(x_pad, s1p, auxp, *wl1)
    rank2d, nmap2d = _rank(score.reshape(npad1 // 128, 128), n1, k1)
    rank = rank2d.reshape(npad1)
    nmap = nmap2d.reshape(npad1)
    x2pad, src2, dst2 = _permute(cs, rank, npad1,
                                 remap=(nmap, src_p, dst_p, n2))
    x2bn, pool1 = _bn_pool(lax.slice(x2pad, (0, 0), (npk1, F)), k1, npk1,
                           params[0]['gamma'][None], params[0]['beta'][None])
    s1p2, auxp2 = _edge_agg(x2bn, src2, dst2, aux, npad2)
    cs2, score2 = _conv_score(x2bn, s1p2, auxp2, *wl2)
    rank2d2, _ = _rank(score2.reshape(npad2 // 128, 128), n2, k2)
    (x3pad,) = _permute(cs2, rank2d2.reshape(npad2), npad2)
    xf_pad, pool2 = _bn_pool(lax.slice(x3pad, (0, 0), (npk2, F)), k2, npk2,
                             params[1]['gamma'][None], params[1]['beta'][None])
    return xf_pad[:k2], jnp.concatenate([pool1, pool2], axis=1)


# speedup vs baseline: 1.1186x; 1.1186x over previous
from math import ceil

import jax
import jax.numpy as jnp
from jax import lax
from jax.experimental import pallas as pl
from jax.experimental.pallas import tpu as pltpu
from jax.experimental.pallas import tpu_sc as plsc

F = 128
_PROBE_LEVEL = 2
_VERBATIM = True
_VAR = 1
NC = 2
NS = 16
NW = NC * NS
EPS = 1e-5
ECH = 128
NCH = 80
BR = 512
IC = 8


def _edge_agg(x_pad, src, dst, aux, n_pad):
    e2 = src.shape[0]
    epw = e2 // NW
    n_chunks = epw // ECH
    rpt = n_pad // NS

    nz = rpt // 64
    probe = _PROBE_LEVEL

    def body(x_hbm, src_hbm, dst_hbm, aux_hbm, z_hbm, za_hbm,
             s1_out, aux_out, srcv, dstv, auxv, rowsv, s1_sh, aux_sh, sem):
        cid = lax.axis_index("c")
        sid = lax.axis_index("s")
        wid = sid * NC + cid
        row0 = sid * rpt
        pltpu.sync_copy(z_hbm, rowsv)
        pltpu.sync_copy(za_hbm, auxv)
        if probe == 0:
            pltpu.sync_copy(rowsv.at[pl.ds(0, 64)],
                            s1_out.at[cid, pl.ds(row0, 64)])
            pltpu.sync_copy(auxv.at[pl.ds(0, 64)],
                            aux_out.at[cid, pl.ds(row0, 64)])
            return
        if probe == 1:
            plsc.subcore_barrier()
            pltpu.sync_copy(rowsv.at[pl.ds(0, 64)],
                            s1_out.at[cid, pl.ds(row0, 64)])
            pltpu.sync_copy(auxv.at[pl.ds(0, 64)],
                            aux_out.at[cid, pl.ds(row0, 64)])
            return
        if probe == 2:
            r2 = sid * 64
            pltpu.sync_copy(rowsv.at[pl.ds(0, 64)],
                            s1_sh.at[pl.ds(r2, 64)])
            pltpu.sync_copy(auxv.at[pl.ds(0, 64)],
                            aux_sh.at[pl.ds(r2, 64)])
            pltpu.sync_copy(s1_sh.at[pl.ds(r2, 64)],
                            rowsv.at[pl.ds(64, 64)])
            pltpu.sync_copy(aux_sh.at[pl.ds(r2, 64)],
                            auxv.at[pl.ds(64, 64)])
            pltpu.sync_copy(rowsv.at[pl.ds(64, 64)],
                            s1_out.at[cid, pl.ds(row0, 64)])
            pltpu.sync_copy(auxv.at[pl.ds(64, 64)],
                            aux_out.at[cid, pl.ds(row0, 64)])
            return

        def zchunk(j, carry):
            r = row0 + j * 64
            pltpu.sync_copy(rowsv.at[pl.ds(0, 64)], s1_sh.at[pl.ds(r, 64)])
            pltpu.sync_copy(auxv.at[pl.ds(0, 64)], aux_sh.at[pl.ds(r, 64)])
            return carry

        lax.fori_loop(0, nz, zchunk, 0)
        plsc.subcore_barrier()
        ebase = wid * epw

        def chunk(j, carry):
            off = ebase + j * ECH
            pltpu.sync_copy(src_hbm.at[pl.ds(off, ECH)], srcv)
            pltpu.sync_copy(dst_hbm.at[pl.ds(off, ECH)], dstv)
            pltpu.sync_copy(aux_hbm.at[pl.ds(off, ECH)], auxv)
            if probe >= 5:
                pltpu.async_copy(x_hbm.at[srcv], rowsv, sem).wait()
            if probe >= 6:
                pltpu.sync_copy(rowsv, s1_sh.at[dstv], add=True)
                pltpu.sync_copy(auxv, aux_sh.at[dstv], add=True)
            return carry

        if probe >= 4:
            lax.fori_loop(0, n_chunks, chunk, 0)
        plsc.subcore_barrier()

        def wchunk(j, carry):
            r = row0 + j * 64
            pltpu.sync_copy(s1_sh.at[pl.ds(r, 64)], rowsv.at[pl.ds(0, 64)])
            pltpu.sync_copy(rowsv.at[pl.ds(0, 64)],
                            s1_out.at[cid, pl.ds(r, 64)])
            pltpu.sync_copy(aux_sh.at[pl.ds(r, 64)], auxv.at[pl.ds(0, 64)])
            pltpu.sync_copy(auxv.at[pl.ds(0, 64)],
                            aux_out.at[cid, pl.ds(r, 64)])
            return carry

        lax.fori_loop(0, nz, wchunk, 0)

    f = pl.kernel(
        body,
        out_type=(jax.ShapeDtypeStruct((NC, n_pad, F), jnp.float32),
                  jax.ShapeDtypeStruct((NC, n_pad, 16), jnp.float32)),
        mesh=plsc.VectorSubcoreMesh(core_axis_name="c", subcore_axis_name="s"),
        scratch_types=[
            pltpu.VMEM((ECH,), jnp.int32),
            pltpu.VMEM((ECH,), jnp.int32),
            pltpu.VMEM((ECH, 16), jnp.float32),
            pltpu.VMEM((ECH, F), jnp.float32),
            pltpu.VMEM_SHARED((n_pad, F), jnp.float32),
            pltpu.VMEM_SHARED((n_pad, 16), jnp.float32),
            pltpu.SemaphoreType.DMA,
        ],
        compiler_params=pltpu.CompilerParams(needs_layout_passes=False),
    )
    zeros = jnp.zeros((ECH, F), jnp.float32)
    zerosa = jnp.zeros((ECH, 16), jnp.float32)
    return f(x_pad, src, dst, aux, zeros, zerosa)


def _conv_score(x_pad, s1p, auxp, wa, wb, we16, w1, w2, b12, wn):
    n_pad = x_pad.shape[0]

    def body(x_ref, s1_ref, aux_ref, wa_ref, wb_ref, we_ref, w1_ref, w2_ref,
             b_ref, wn_ref, cs_ref, sc_ref):
        x = x_ref[...]
        s1 = s1_ref[0] + s1_ref[1]
        aux = aux_ref[0] + aux_ref[1]
        cnt = aux[:, 4:5]
        num = (cnt * jnp.dot(x, wa_ref[...], preferred_element_type=jnp.float32)
               + jnp.dot(s1, wb_ref[...], preferred_element_type=jnp.float32)
               + jnp.dot(aux, we_ref[...], preferred_element_type=jnp.float32))
        mean = num / jnp.maximum(cnt, 1.0)
        conv = (jnp.dot(x, w1_ref[...], preferred_element_type=jnp.float32)
                + jnp.dot(mean, w2_ref[...], preferred_element_type=jnp.float32)
                + b_ref[...])
        arg = jnp.sum(conv * wn_ref[...], axis=1, keepdims=True)
        score = 1.0 / (1.0 + jnp.exp(-arg))
        cs_ref[...] = conv * score
        sc_ref[...] = score

    wspec = pl.BlockSpec((F, F), lambda i: (0, 0))
    vspec = pl.BlockSpec((1, F), lambda i: (0, 0))
    return pl.pallas_call(
        body,
        grid=(n_pad // BR,),
        in_specs=[
            pl.BlockSpec((BR, F), lambda i: (i, 0)),
            pl.BlockSpec((NC, BR, F), lambda i: (0, i, 0)),
            pl.BlockSpec((NC, BR, 16), lambda i: (0, i, 0)),
            wspec, wspec,
            pl.BlockSpec((16, F), lambda i: (0, 0)),
            wspec, wspec, vspec, vspec,
        ],
        out_specs=[pl.BlockSpec((BR, F), lambda i: (i, 0)),
                   pl.BlockSpec((BR, 1), lambda i: (i, 0))],
        out_shape=[jax.ShapeDtypeStruct((n_pad, F), jnp.float32),
                   jax.ShapeDtypeStruct((n_pad, 1), jnp.float32)],
    )(x_pad, s1p, auxp, wa, wb, we16, w1, w2, b12, wn)


def _rank(score2d, n_valid, k):
    g = score2d.shape[0]

    def body(s_ref, rank_ref, nmap_ref):
        i0 = pl.program_id(0) * IC
        si_raw = s_ref[pl.ds(i0, IC), :]
        iival = ((i0 + lax.broadcasted_iota(jnp.int32, (IC, 128), 0)) * 128
                 + lax.broadcasted_iota(jnp.int32, (IC, 128), 1))
        si = jnp.where(iival < n_valid, si_raw, -jnp.inf)[:, :, None]
        ii = iival[:, :, None]

        def jstep(jc, acc):
            sj_raw = s_ref[pl.ds(jc, 1), :][None]
            jj = (jc * 128
                  + lax.broadcasted_iota(jnp.int32, (1, 1, 128), 2))
            sj = jnp.where(jj < n_valid, sj_raw, -jnp.inf)
            cmp = (sj > si) | ((sj == si) & (jj < ii))
            return acc + jnp.sum(cmp.astype(jnp.int32), axis=2)

        acc = lax.fori_loop(0, g, jstep, jnp.zeros((IC, 128), jnp.int32))
        valid_i = iival < n_valid
        rank = jnp.where(valid_i, acc, iival)
        rank_ref[...] = rank
        nmap_ref[...] = jnp.where(valid_i & (rank < k), rank, -1)

    return pl.pallas_call(
        body,
        grid=(g // IC,),
        in_specs=[pl.BlockSpec((g, 128), lambda i: (0, 0))],
        out_specs=[pl.BlockSpec((IC, 128), lambda i: (i, 0)),
                   pl.BlockSpec((IC, 128), lambda i: (i, 0))],
        out_shape=[jax.ShapeDtypeStruct((g, 128), jnp.int32),
                   jax.ShapeDtypeStruct((g, 128), jnp.int32)],
    )(score2d)


def _permute(cs, rank, n_pad, npk, remap=None):
    npt = n_pad // NW
    n_iters = npt // NCH
    zrows = npk // NS
    zi = zrows // NCH
    do_remap = remap is not None
    if do_remap:
        nmap, src, dst, n_new = remap
        e2 = src.shape[0]
        epw = e2 // NW
        e_chunks = epw // ECH

    def body(*refs):
        if do_remap:
            (cs_hbm, rank_hbm, z_hbm, nmap_hbm, src_hbm, dst_hbm,
             xout, src2, dst2,
             rowsv, rkv, x_sh, nmapv, srcv, dstv, osrcv, odstv, sem) = refs
        else:
            cs_hbm, rank_hbm, z_hbm, xout, rowsv, rkv, x_sh, sem = refs
        cid = lax.axis_index("c")
        sid = lax.axis_index("s")
        wid = sid * NC + cid
        nb0 = wid * npt
        zr0 = sid * zrows
        pltpu.sync_copy(z_hbm, rowsv)

        def zchunk(j, carry):
            pltpu.sync_copy(rowsv, x_sh.at[pl.ds(zr0 + j * NCH, NCH)])
            return carry

        lax.fori_loop(0, zi, zchunk, 0)
        plsc.subcore_barrier()

        def pchunk(j, carry):
            nb = nb0 + j * NCH
            pltpu.sync_copy(cs_hbm.at[pl.ds(nb, NCH)], rowsv)
            pltpu.sync_copy(rank_hbm.at[pl.ds(nb, NCH)], rkv)
            pltpu.sync_copy(rowsv, x_sh.at[rkv], add=True)
            return carry

        lax.fori_loop(0, n_iters, pchunk, 0)
        plsc.subcore_barrier()

        def wchunk(j, carry):
            r = zr0 + j * NCH
            pltpu.sync_copy(x_sh.at[pl.ds(r, NCH)], rowsv)
            pltpu.sync_copy(rowsv, xout.at[cid, pl.ds(r, NCH)])
            return carry

        lax.fori_loop(0, zi, wchunk, 0)

        if do_remap:
            pltpu.sync_copy(nmap_hbm, nmapv)
            ebase = wid * epw

            def echunk(j, carry):
                off = ebase + j * ECH
                pltpu.sync_copy(src_hbm.at[pl.ds(off, ECH)], srcv)
                pltpu.sync_copy(dst_hbm.at[pl.ds(off, ECH)], dstv)
                for t in range(ECH // 16):
                    sl = pl.ds(t * 16, 16)
                    a = plsc.load_gather(nmapv, [srcv[sl]])
                    b = plsc.load_gather(nmapv, [dstv[sl]])
                    ok = (a >= 0) & (b >= 0)
                    gid = off + t * 16 + lax.iota(jnp.int32, 16)
                    osrcv[sl] = jnp.where(ok, a, gid & 4095)
                    odstv[sl] = jnp.where(ok, b, n_new + (gid & 63))
                pltpu.sync_copy(osrcv, src2.at[pl.ds(off, ECH)])
                pltpu.sync_copy(odstv, dst2.at[pl.ds(off, ECH)])
                return carry

            lax.fori_loop(0, e_chunks, echunk, 0)

    out_type = [jax.ShapeDtypeStruct((NC, npk, F), jnp.float32)]
    scratch = [
        pltpu.VMEM((NCH, F), jnp.float32),
        pltpu.VMEM((NCH,), jnp.int32),
        pltpu.VMEM_SHARED((n_pad, F), jnp.float32),
    ]
    args = [cs, rank, jnp.zeros((NCH, F), jnp.float32)]
    if do_remap:
        out_type += [jax.ShapeDtypeStruct((e2,), jnp.int32),
                     jax.ShapeDtypeStruct((e2,), jnp.int32)]
        scratch += [
            pltpu.VMEM((n_pad,), jnp.int32),
            pltpu.VMEM((ECH,), jnp.int32),
            pltpu.VMEM((ECH,), jnp.int32),
            pltpu.VMEM((ECH,), jnp.int32),
            pltpu.VMEM((ECH,), jnp.int32),
        ]
        args += [nmap, src, dst]
    scratch += [pltpu.SemaphoreType.DMA]

    f = pl.kernel(
        body,
        out_type=tuple(out_type),
        mesh=plsc.VectorSubcoreMesh(core_axis_name="c", subcore_axis_name="s"),
        scratch_types=scratch,
        compiler_params=pltpu.CompilerParams(needs_layout_passes=False),
    )
    return f(*args)


def _bn_pool(x_sel, kk, npk, gamma, beta):
    def body(x_ref, g_ref, b_ref, xbn_ref, pool_ref):
        xp = x_ref[0] + x_ref[1]
        m = lax.broadcasted_iota(jnp.int32, (npk, 1), 0) < kk
        xm = jnp.where(m, xp, 0.0)
        mu = jnp.sum(xm, axis=0, keepdims=True) / kk
        var = jnp.sum(xm * xm, axis=0, keepdims=True) / kk - mu * mu
        inv = lax.rsqrt(var + EPS)
        xn = jnp.maximum((xp - mu) * inv * g_ref[...] + b_ref[...], 0.0)
        xn = jnp.where(m, xn, 0.0)
        xbn_ref[...] = xn
        mx = jnp.max(jnp.where(m, xn, -jnp.inf), axis=0, keepdims=True)
        mn = jnp.sum(xn, axis=0, keepdims=True) / kk
        pool_ref[...] = jnp.concatenate([mx, mn], axis=1)

    return pl.pallas_call(
        body,
        in_specs=[pl.BlockSpec((NC, npk, F), lambda: (0, 0, 0)),
                  pl.BlockSpec((1, F), lambda: (0, 0)),
                  pl.BlockSpec((1, F), lambda: (0, 0))],
        out_specs=[pl.BlockSpec((npk, F), lambda: (0, 0)),
                   pl.BlockSpec((1, 2 * F), lambda: (0, 0))],
        out_shape=[jax.ShapeDtypeStruct((npk, F), jnp.float32),
                   jax.ShapeDtypeStruct((1, 2 * F), jnp.float32)],
    )(x_sel, gamma, beta)


def _gather2(xtab, srcg, dstg):
    e = srcg.shape[0]
    epw = e // NW
    nfull = epw // ECH
    tail = epw - nfull * ECH

    def body(x_hbm, src_hbm, dst_hbm, xd_out, xs_out,
             srcv, dstv, rs, rd, srcv16, dstv16, r16a, r16b, sem, sem2):
        cid = lax.axis_index("c")
        sid = lax.axis_index("s")
        wid = sid * NC + cid
        ebase = wid * epw

        def chunk(j, c):
            off = ebase + j * ECH
            pltpu.sync_copy(src_hbm.at[pl.ds(off, ECH)], srcv)
            pltpu.sync_copy(dst_hbm.at[pl.ds(off, ECH)], dstv)
            pltpu.async_copy(x_hbm.at[srcv], rs, sem).wait()
            pltpu.async_copy(x_hbm.at[dstv], rd, sem2).wait()
            pltpu.sync_copy(rs, xs_out.at[pl.ds(off, ECH)])
            pltpu.sync_copy(rd, xd_out.at[pl.ds(off, ECH)])
            return c

        lax.fori_loop(0, nfull, chunk, 0)
        if tail:
            off = ebase + nfull * ECH
            pltpu.sync_copy(src_hbm.at[pl.ds(off, tail)], srcv16)
            pltpu.sync_copy(dst_hbm.at[pl.ds(off, tail)], dstv16)
            pltpu.async_copy(x_hbm.at[srcv16], r16a, sem).wait()
            pltpu.async_copy(x_hbm.at[dstv16], r16b, sem2).wait()
            pltpu.sync_copy(r16a, xs_out.at[pl.ds(off, tail)])
            pltpu.sync_copy(r16b, xd_out.at[pl.ds(off, tail)])

    f = pl.kernel(
        body,
        out_type=(jax.ShapeDtypeStruct((e, F), jnp.float32),
                  jax.ShapeDtypeStruct((e, F), jnp.float32)),
        mesh=plsc.VectorSubcoreMesh(core_axis_name="c", subcore_axis_name="s"),
        scratch_types=[
            pltpu.VMEM((ECH,), jnp.int32),
            pltpu.VMEM((ECH,), jnp.int32),
            pltpu.VMEM((ECH, F), jnp.float32),
            pltpu.VMEM((ECH, F), jnp.float32),
            pltpu.VMEM((max(tail, 8),), jnp.int32),
            pltpu.VMEM((max(tail, 8),), jnp.int32),
            pltpu.VMEM((max(tail, 8), F), jnp.float32),
            pltpu.VMEM((max(tail, 8), F), jnp.float32),
            pltpu.SemaphoreType.DMA,
            pltpu.SemaphoreType.DMA,
        ],
        compiler_params=pltpu.CompilerParams(needs_layout_passes=False),
    )
    return f(xtab, srcg, dstg)


def _mm_cat(xd, xs, er, w, b, br=512):
    n = xd.shape[0]
    fo = w.shape[1]

    def body(xd_ref, xs_ref, er_ref, w_ref, b_ref, o_ref):
        h = jnp.concatenate([xd_ref[...], xs_ref[...], er_ref[...]], axis=1)
        o_ref[...] = (jnp.dot(h, w_ref[...],
                              preferred_element_type=jnp.float32)
                      + b_ref[...])

    espec = pl.BlockSpec((br, F), lambda i: (i, 0))
    return pl.pallas_call(
        body,
        grid=(n // br,),
        in_specs=[espec, espec, espec,
                  pl.BlockSpec((3 * F, fo), lambda i: (0, 0)),
                  pl.BlockSpec((1, fo), lambda i: (0, 0))],
        out_specs=pl.BlockSpec((br, fo), lambda i: (i, 0)),
        out_shape=jax.ShapeDtypeStruct((n, fo), jnp.float32),
    )(xd, xs, er, w, b[None])


def _mm_bias(h, w, b, br=512):
    n, kdim = h.shape
    fo = w.shape[1]

    def body(h_ref, w_ref, b_ref, o_ref):
        o_ref[...] = (jnp.dot(h_ref[...], w_ref[...],
                              preferred_element_type=jnp.float32)
                      + b_ref[...])

    return pl.pallas_call(
        body,
        grid=(n // br,),
        in_specs=[pl.BlockSpec((br, kdim), lambda i: (i, 0)),
                  pl.BlockSpec((kdim, fo), lambda i: (0, 0)),
                  pl.BlockSpec((1, fo), lambda i: (0, 0))],
        out_specs=pl.BlockSpec((br, fo), lambda i: (i, 0)),
        out_shape=jax.ShapeDtypeStruct((n, fo), jnp.float32),
    )(h, w, b[None])


def _fold_weights(p):
    wa = p['Wpre'][0:F]
    wb = p['Wpre'][F:2 * F]
    c = p['Wpre'][2 * F:3 * F]
    we16 = (jnp.zeros((16, F), jnp.float32)
            .at[0:4].set(p['We'] @ c)
            .at[4].set(p['be'] @ c + p['bpre']))
    w1 = p['Wpost'][:F] @ p['Wlin']
    w2 = p['Wpost'][F:] @ p['Wlin']
    b12 = (p['bpost'] @ p['Wlin'] + p['blin'])[None]
    wn = (p['wpool'] / jnp.linalg.norm(p['wpool']))[None]
    return wa, wb, we16, w1, w2, b12, wn


def _round_up(v, m):
    return ((v + m - 1) // m) * m


def kernel(x, edge_index, edge_attr, batch, params):
    n1 = x.shape[0]
    e = edge_index.shape[1]
    k1 = int(ceil(0.5 * n1))
    n2 = k1
    k2 = int(ceil(0.5 * n2))
    npad1 = _round_up(n1 + 64, NW * NCH)
    npad2 = _round_up(n2 + 64, NW * NCH)
    npk1 = _round_up(k1, NW * NCH)
    npk2 = _round_up(k2, NW * NCH)
    e2 = _round_up(e, NW * ECH)

    src = edge_index[0]
    dst = edge_index[1]
    pad_e = e2 - e
    eid = jnp.arange(pad_e, dtype=jnp.int32)
    src_p = jnp.concatenate([src, eid & 4095])
    dst_p = jnp.concatenate([dst, n1 + (eid & 63)])
    aux = (jnp.zeros((e2, 16), jnp.float32)
           .at[:e, :4].set(edge_attr)
           .at[:e, 4].set(1.0))
    x_pad = jnp.zeros((npad1, F), jnp.float32).at[:n1].set(x)

    wl1 = _fold_weights(params[0])
    wl2 = _fold_weights(params[1])

    if _VERBATIM:
        nn = n1
        emask = jnp.ones((e,), bool)
        srcr = src
        dstr = dst
        xr = x
        pooled = []
        for p in params:
            wa_, wb_, wc_ = (p['Wpre'][0:F], p['Wpre'][F:2 * F],
                             p['Wpre'][2 * F:3 * F])
            er = edge_attr @ p['We'] + p['be']
            if _VAR >= 3:
                dst_eff = jnp.where(emask, dstr, nn)
                msk = emask.astype(xr.dtype)[:, None]
                s1_ = jnp.zeros((nn + 1, F), xr.dtype).at[dst_eff].add(
                    xr[srcr] * msk)
                se_ = jnp.zeros((nn + 1, F), xr.dtype).at[dst_eff].add(
                    er * msk)
                cnt_ = jnp.zeros((nn + 1,), xr.dtype).at[dst_eff].add(
                    emask.astype(xr.dtype))
                cc = cnt_[:nn][:, None]
                sums = (cc * (xr @ wa_) + s1_[:nn] @ wb_ + se_[:nn] @ wc_
                        + cc * p['bpre'])
                mean = sums / jnp.maximum(cc, 1.0)
            elif _VAR >= 2:
                m = (xr[dstr] @ wa_ + xr[srcr] @ wb_ + er @ wc_
                     + p['bpre'])
                dst_eff = jnp.where(emask, dstr, nn)
                sums = jnp.zeros((nn + 1, F), xr.dtype).at[dst_eff].add(m)
                cnt_ = jnp.zeros((nn + 1,), xr.dtype).at[dst_eff].add(1.0)
                mean = sums[:nn] / jnp.maximum(cnt_[:nn], 1.0)[:, None]
            else:
                eidv = jnp.arange(e, dtype=jnp.int32)
                srcg = jnp.where(emask, srcr, eidv & 2047)
                dstg = jnp.where(emask, dstr, eidv & 2047)
                xd_g, xs_g = _gather2(xr, srcg, dstg)
                m = _mm_cat(xd_g, xs_g, er, p['Wpre'], p['bpre'])
                dst_eff = jnp.where(emask, dstr, nn)
                sums = jnp.zeros((nn + 1, F), xr.dtype).at[dst_eff].add(m)
                cnt_ = jnp.zeros((nn + 1,), xr.dtype).at[dst_eff].add(1.0)
                mean = sums[:nn] / jnp.maximum(cnt_[:nn], 1.0)[:, None]
            if _VAR >= 4:
                outv = xr @ p['Wpost'][:F] + mean @ p['Wpost'][F:] + p['bpost']
            else:
                outv = jnp.concatenate([xr, mean], axis=-1) @ p['Wpost'] + p['bpost']
            conv = outv @ p['Wlin'] + p['blin']
            score = jax.nn.sigmoid((conv * p['wpool']).sum(-1)
                                   / jnp.linalg.norm(p['wpool']))
            kk = n2 if nn == n1 else k2
            perm = jnp.argsort(-score)[:kk]
            xp = conv[perm] * score[perm][:, None]
            nmaskr = jnp.zeros((nn,), bool).at[perm].set(True)
            nmapr = jnp.zeros((nn,), jnp.int32).at[perm].set(
                jnp.arange(kk, dtype=jnp.int32))
            emask = emask & nmaskr[srcr] & nmaskr[dstr]
            srcr = jnp.where(emask, nmapr[srcr], 0)
            dstr = jnp.where(emask, nmapr[dstr], 0)
            mu = xp.mean(0)
            var = xp.var(0)
            xr = jax.nn.relu((xp - mu) / jnp.sqrt(var + EPS) * p['gamma']
                             + p['beta'])
            nn = kk
            pooled.append(jnp.concatenate(
                [xr.max(0, keepdims=True), xr.mean(0, keepdims=True)],
                axis=1))
        return xr, jnp.concatenate(pooled, axis=1)

    def jlayer(xc, s1p_, auxp_, w, p, nn, kk):
        wa, wb, we16, w1, w2, b12, wn = w
        s1 = (s1p_[0] + s1p_[1])[:nn]
        auxs = (auxp_[0] + auxp_[1])[:nn]
        cnt = auxs[:, 4:5]
        num = cnt * (xc[:nn] @ wa) + s1 @ wb + auxs @ we16
        mean = num / jnp.maximum(cnt, 1.0)
        conv = xc[:nn] @ w1 + mean @ w2 + b12
        score = jax.nn.sigmoid(conv @ wn[0])
        perm = jnp.argsort(-score)[:kk]
        xp = conv[perm] * score[perm][:, None]
        nmask = jnp.zeros((nn,), bool).at[perm].set(True)
        nmapj = jnp.zeros((nn,), jnp.int32).at[perm].set(
            jnp.arange(kk, dtype=jnp.int32))
        mu = xp.mean(0)
        var = xp.var(0)
        xn = jax.nn.relu((xp - mu) / jnp.sqrt(var + EPS) * p['gamma']
                         + p['beta'])
        pooled = jnp.concatenate(
            [xn.max(0, keepdims=True), xn.mean(0, keepdims=True)], axis=1)
        return xn, nmask, nmapj, pooled

    def agg(xc, srcc, dstc, np_):
        s1p_, auxp_ = _edge_agg(xc, srcc, dstc, aux, np_)
        if _PROBE_LEVEL < 6:
            s1j = (jnp.zeros((np_, F)).at[dstc].add(xc[srcc])
                   + 0.0 * jnp.nan_to_num(s1p_[0], posinf=0.0, neginf=0.0))
            auxj = (jnp.zeros((np_, 16)).at[dstc].add(aux)
                    + 0.0 * jnp.nan_to_num(auxp_[0], posinf=0.0, neginf=0.0))
            s1p_ = jnp.stack([s1j, jnp.zeros_like(s1j)])
            auxp_ = jnp.stack([auxj, jnp.zeros_like(auxj)])
        return s1p_, auxp_

    s1p, auxp = agg(x_pad, src_p, dst_p, npad1)
    x1j, nmask, nmapj, pool1 = jlayer(x_pad, s1p, auxp, wl1, params[0],
                                      n1, k1)
    okj = (dst_p < n1)
    vs = jnp.where(okj, src_p, 0)
    vd = jnp.where(okj, dst_p, 0)
    ok2 = okj & nmask[vs] & nmask[vd]
    src2j = jnp.where(ok2, nmapj[vs], jnp.arange(e2, dtype=jnp.int32) & 4095)
    dst2j = jnp.where(ok2, nmapj[vd],
                      n2 + (jnp.arange(e2, dtype=jnp.int32) & 63))
    x2j = jnp.zeros((npad2, F), jnp.float32).at[:k1].set(x1j)
    s1p2, auxp2 = agg(x2j, src2j, dst2j, npad2)
    x2jj, _, _, pool2 = jlayer(x2j, s1p2, auxp2, wl2, params[1], n2, k2)
    return x2jj, jnp.concatenate([pool1, pool2], axis=1)

    s1p, auxp = _edge_agg(x_pad, src_p, dst_p, aux, npad1)
    cs, score = _conv_score(x_pad, s1p, auxp, *wl1)
    rank2d, nmap2d = _rank(score.reshape(npad1 // 128, 128), n1, k1)
    rank = rank2d.reshape(npad1)
    nmap = nmap2d.reshape(npad1)
    x2parts, src2, dst2 = _permute(cs, rank, npad1, npk1,
                                   remap=(nmap, src_p, dst_p, n2))
    x2bn, pool1 = _bn_pool(x2parts, k1, npk1,
                           params[0]['gamma'][None], params[0]['beta'][None])
    s1p2, auxp2 = _edge_agg(x2bn, src2, dst2, aux, npad2)
    cs2, score2 = _conv_score(x2bn, s1p2, auxp2, *wl2)
    rank2d2, _ = _rank(score2.reshape(npad2 // 128, 128), n2, k2)
    (x3parts,) = _permute(cs2, rank2d2.reshape(npad2), npad2, npk2)
    xf_pad, pool2 = _bn_pool(x3parts, k2, npk2,
                             params[1]['gamma'][None], params[1]['beta'][None])
    return xf_pad[:k2], jnp.concatenate([pool1, pool2], axis=1)


# speedup vs baseline: 1.1192x; 1.0005x over previous
import jax
import jax.numpy as jnp
from jax import lax
from jax.experimental import pallas as pl
from jax.experimental.pallas import tpu as pltpu
from jax.experimental.pallas import tpu_sc as plsc

F = 128
NC = 2
NS = 16
NW = NC * NS
EPS = 1e-5
ECH = 128


def _gather2(xtab, srcg, dstg):
    e = srcg.shape[0]
    epw = e // NW
    nfull = epw // ECH
    tail = epw - nfull * ECH

    def body(x_hbm, src_hbm, dst_hbm, xd_out, xs_out,
             srcv, dstv, rs, rd, srcv16, dstv16, r16a, r16b, sem, sem2):
        cid = lax.axis_index("c")
        sid = lax.axis_index("s")
        wid = sid * NC + cid
        ebase = wid * epw

        def chunk(j, c):
            off = ebase + j * ECH
            pltpu.sync_copy(src_hbm.at[pl.ds(off, ECH)], srcv)
            pltpu.sync_copy(dst_hbm.at[pl.ds(off, ECH)], dstv)
            pltpu.async_copy(x_hbm.at[srcv], rs, sem).wait()
            pltpu.async_copy(x_hbm.at[dstv], rd, sem2).wait()
            pltpu.sync_copy(rs, xs_out.at[pl.ds(off, ECH)])
            pltpu.sync_copy(rd, xd_out.at[pl.ds(off, ECH)])
            return c

        lax.fori_loop(0, nfull, chunk, 0)
        if tail:
            off = ebase + nfull * ECH
            pltpu.sync_copy(src_hbm.at[pl.ds(off, tail)], srcv16)
            pltpu.sync_copy(dst_hbm.at[pl.ds(off, tail)], dstv16)
            pltpu.async_copy(x_hbm.at[srcv16], r16a, sem).wait()
            pltpu.async_copy(x_hbm.at[dstv16], r16b, sem2).wait()
            pltpu.sync_copy(r16a, xs_out.at[pl.ds(off, tail)])
            pltpu.sync_copy(r16b, xd_out.at[pl.ds(off, tail)])

    f = pl.kernel(
        body,
        out_type=(jax.ShapeDtypeStruct((e, F), jnp.float32),
                  jax.ShapeDtypeStruct((e, F), jnp.float32)),
        mesh=plsc.VectorSubcoreMesh(core_axis_name="c", subcore_axis_name="s"),
        scratch_types=[
            pltpu.VMEM((ECH,), jnp.int32),
            pltpu.VMEM((ECH,), jnp.int32),
            pltpu.VMEM((ECH, F), jnp.float32),
            pltpu.VMEM((ECH, F), jnp.float32),
            pltpu.VMEM((max(tail, 8),), jnp.int32),
            pltpu.VMEM((max(tail, 8),), jnp.int32),
            pltpu.VMEM((max(tail, 8), F), jnp.float32),
            pltpu.VMEM((max(tail, 8), F), jnp.float32),
            pltpu.SemaphoreType.DMA,
            pltpu.SemaphoreType.DMA,
        ],
        compiler_params=pltpu.CompilerParams(needs_layout_passes=False),
    )
    return f(xtab, srcg, dstg)


def _mm_cat(xd, xs, er, w, b, br=512):
    n = xd.shape[0]
    fo = w.shape[1]

    def body(xd_ref, xs_ref, er_ref, w_ref, b_ref, o_ref):
        h = jnp.concatenate([xd_ref[...], xs_ref[...], er_ref[...]], axis=1)
        o_ref[...] = (jnp.dot(h, w_ref[...],
                              preferred_element_type=jnp.float32)
                      + b_ref[...])

    espec = pl.BlockSpec((br, F), lambda i: (i, 0))
    return pl.pallas_call(
        body,
        grid=(n // br,),
        in_specs=[espec, espec, espec,
                  pl.BlockSpec((3 * F, fo), lambda i: (0, 0)),
                  pl.BlockSpec((1, fo), lambda i: (0, 0))],
        out_specs=pl.BlockSpec((br, fo), lambda i: (i, 0)),
        out_shape=jax.ShapeDtypeStruct((n, fo), jnp.float32),
    )(xd, xs, er, w, b[None])


def kernel(x, edge_index, edge_attr, batch, params):
    del batch
    n1 = x.shape[0]
    e = edge_index.shape[1]
    src = edge_index[0]
    dst = edge_index[1]

    nn = n1
    emask = jnp.ones((e,), bool)
    srcr = src
    dstr = dst
    xr = x
    pooled = []
    eidv = jnp.arange(e, dtype=jnp.int32)
    for p in params:
        er = edge_attr @ p['We'] + p['be']
        srcg = jnp.where(emask, srcr, eidv & 2047)
        dstg = jnp.where(emask, dstr, eidv & 2047)
        xd_g, xs_g = _gather2(xr, srcg, dstg)
        m = _mm_cat(xd_g, xs_g, er, p['Wpre'], p['bpre'])
        dst_eff = jnp.where(emask, dstr, nn)
        sums = jnp.zeros((nn + 1, F), xr.dtype).at[dst_eff].add(m)
        cnt_ = jnp.zeros((nn + 1,), xr.dtype).at[dst_eff].add(1.0)
        mean = sums[:nn] / jnp.maximum(cnt_[:nn], 1.0)[:, None]
        outv = jnp.concatenate([xr, mean], axis=-1) @ p['Wpost'] + p['bpost']
        conv = outv @ p['Wlin'] + p['blin']
        score = jax.nn.sigmoid((conv * p['wpool']).sum(-1)
                               / jnp.linalg.norm(p['wpool']))
        kk = (nn + 1) // 2
        perm = jnp.argsort(-score)[:kk]
        xp = conv[perm] * score[perm][:, None]
        nmaskr = jnp.zeros((nn,), bool).at[perm].set(True)
        nmapr = jnp.zeros((nn,), jnp.int32).at[perm].set(
            jnp.arange(kk, dtype=jnp.int32))
        emask = emask & nmaskr[srcr] & nmaskr[dstr]
        srcr = jnp.where(emask, nmapr[srcr], 0)
        dstr = jnp.where(emask, nmapr[dstr], 0)
        mu = xp.mean(0)
        var = xp.var(0)
        xr = jax.nn.relu((xp - mu) / jnp.sqrt(var + EPS) * p['gamma']
                         + p['beta'])
        nn = kk
        pooled.append(jnp.concatenate(
            [xr.max(0, keepdims=True), xr.mean(0, keepdims=True)], axis=1))
    return xr, jnp.concatenate(pooled, axis=1)
